# Initial kernel scaffold; baseline (speedup 1.0000x reference)
import jax
import jax.numpy as jnp
from jax.experimental import pallas as pl


def kernel(x, edge_index, W1, att_src1, att_dst1, b1, W2, att_src2, att_dst2, b2):
    raise NotImplementedError("write your pallas kernel here")


# speedup vs baseline: 23.2909x; 23.2909x over previous
import functools

import jax
import jax.numpy as jnp
from jax import lax
from jax.experimental import pallas as pl
from jax.experimental.pallas import tpu as pltpu
from jax.experimental.pallas import tpu_sc as plsc

N = 10000
E = 320000
D = 128

NC = 2
NS = 16
NW = NC * NS
EP = E // NW
K = 80
NCHUNK = EP // K
NP = 10240
RPT = NP // NS
L = 16

_EPS = 1e-16


def _mm_body(x_ref, w_ref, o_ref):
    o_ref[...] = jnp.dot(x_ref[...], w_ref[...],
                         preferred_element_type=jnp.float32)


def _matmul_ext(x, w_ext, bm=1000):
    m = x.shape[0]
    grid = (m // bm,)
    return pl.pallas_call(
        _mm_body,
        grid=grid,
        in_specs=[
            pl.BlockSpec((bm, D), lambda i: (i, 0)),
            pl.BlockSpec((D, 2 * D), lambda i: (0, 0)),
        ],
        out_specs=pl.BlockSpec((bm, 2 * D), lambda i: (i, 0)),
        out_shape=jax.ShapeDtypeStruct((m, 2 * D), jnp.float32),
    )(x, w_ext)


def _comb_mm_body(a0_ref, a1_ref, d0_ref, d1_ref, b_ref, w_ref, o_ref):
    den = d0_ref[...] + d1_ref[...]
    h = (a0_ref[...] + a1_ref[...]) / (den + _EPS) + b_ref[...]
    h = jnp.maximum(h, 0.0)
    o_ref[...] = jnp.dot(h, w_ref[...], preferred_element_type=jnp.float32)


def _combine_matmul(a0, a1, d0, d1, b, w_ext, bm=1000):
    m = a0.shape[0]
    grid = (m // bm,)
    return pl.pallas_call(
        _comb_mm_body,
        grid=grid,
        in_specs=[
            pl.BlockSpec((bm, D), lambda i: (i, 0)),
            pl.BlockSpec((bm, D), lambda i: (i, 0)),
            pl.BlockSpec((bm, 1), lambda i: (i, 0)),
            pl.BlockSpec((bm, 1), lambda i: (i, 0)),
            pl.BlockSpec((1, D), lambda i: (0, 0)),
            pl.BlockSpec((D, 2 * D), lambda i: (0, 0)),
        ],
        out_specs=pl.BlockSpec((bm, 2 * D), lambda i: (i, 0)),
        out_shape=jax.ShapeDtypeStruct((m, 2 * D), jnp.float32),
    )(a0, a1, d0, d1, b, w_ext)


def _comb_body(a0_ref, a1_ref, d0_ref, d1_ref, b_ref, o_ref):
    den = d0_ref[...] + d1_ref[...]
    h = (a0_ref[...] + a1_ref[...]) / (den + _EPS) + b_ref[...]
    o_ref[...] = jnp.maximum(h, 0.0)


def _combine(a0, a1, d0, d1, b, bm=1000):
    m = a0.shape[0]
    grid = (m // bm,)
    return pl.pallas_call(
        _comb_body,
        grid=grid,
        in_specs=[
            pl.BlockSpec((bm, D), lambda i: (i, 0)),
            pl.BlockSpec((bm, D), lambda i: (i, 0)),
            pl.BlockSpec((bm, 1), lambda i: (i, 0)),
            pl.BlockSpec((bm, 1), lambda i: (i, 0)),
            pl.BlockSpec((1, D), lambda i: (0, 0)),
        ],
        out_specs=pl.BlockSpec((bm, D), lambda i: (i, 0)),
        out_shape=jax.ShapeDtypeStruct((m, D), jnp.float32),
    )(a0, a1, d0, d1, b)


def _edge_body(xp_hbm, asrc_hbm, adst_hbm, src_hbm, dst_hbm,
               acc_out, den_out,
               idx_s, idx_d, a_s, a_d, wbuf, rows, dzero,
               asrc_sh, adst_sh, acc_sh, den_sh, sem):
    c = lax.axis_index("c")
    s = lax.axis_index("s")

    @pl.when(s == 0)
    def _stage():
        pltpu.sync_copy(asrc_hbm, asrc_sh)
        pltpu.sync_copy(adst_hbm, adst_sh)

    def _zrow(r, _):
        for j in range(D // L):
            rows[r, pl.ds(j * L, L)] = jnp.zeros((L,), jnp.float32)
        return 0
    lax.fori_loop(0, K, _zrow, 0)
    def _zd(r, _):
        dzero[pl.ds(r * L, L)] = jnp.zeros((L,), jnp.float32)
        return 0
    lax.fori_loop(0, RPT // L, _zd, 0)

    base_r = s * RPT
    for j in range(RPT // K):
        pltpu.sync_copy(rows, acc_sh.at[pl.ds(base_r + j * K, K)])
    pltpu.sync_copy(dzero, den_sh.at[pl.ds(base_r, RPT)])
    plsc.subcore_barrier()

    wid = s * NC + c
    ebase = wid * EP

    def _chunk(i, _):
        b = ebase + i * K
        pltpu.sync_copy(src_hbm.at[pl.ds(b, K)], idx_s)
        pltpu.sync_copy(dst_hbm.at[pl.ds(b, K)], idx_d)
        cp = pltpu.async_copy(xp_hbm.at[idx_s], rows, sem)
        pltpu.sync_copy(asrc_sh.at[idx_s], a_s)
        pltpu.sync_copy(adst_sh.at[idx_d], a_d)
        for j in range(K // L):
            al = a_s[pl.ds(j * L, L)] + a_d[pl.ds(j * L, L)]
            al = jnp.where(al >= 0.0, al, 0.2 * al)
            wbuf[pl.ds(j * L, L)] = jnp.exp(al)
        cp.wait()
        def _scale(g, _):
            wv = wbuf[pl.ds(g * L, L)]
            r0 = g * L
            for i in range(L):
                wvi = jnp.full((L,), wv[i], jnp.float32)
                for jj in range(D // L):
                    rows[r0 + i, pl.ds(jj * L, L)] = (
                        rows[r0 + i, pl.ds(jj * L, L)] * wvi)
            return 0
        lax.fori_loop(0, K // L, _scale, 0)
        pltpu.sync_copy(rows, acc_sh.at[idx_d], add=True)
        pltpu.sync_copy(wbuf, den_sh.at[idx_d], add=True)
        return 0

    lax.fori_loop(0, NCHUNK, _chunk, 0)
    plsc.subcore_barrier()

    pltpu.sync_copy(acc_sh.at[pl.ds(base_r, RPT)],
                    acc_out.at[c, pl.ds(base_r, RPT)])
    pltpu.sync_copy(den_sh.at[pl.ds(base_r, RPT)],
                    den_out.at[c, pl.ds(base_r, RPT)])


def _edge_pass(xp, a_src, a_dst, src, dst):
    mesh = plsc.VectorSubcoreMesh(core_axis_name="c", subcore_axis_name="s")
    fn = pl.kernel(
        _edge_body,
        out_type=[
            jax.ShapeDtypeStruct((NC, NP, D), jnp.float32),
            jax.ShapeDtypeStruct((NC, NP), jnp.float32),
        ],
        mesh=mesh,
        scratch_types=[
            pltpu.VMEM((K,), jnp.int32),
            pltpu.VMEM((K,), jnp.int32),
            pltpu.VMEM((K,), jnp.float32),
            pltpu.VMEM((K,), jnp.float32),
            pltpu.VMEM((K,), jnp.float32),
            pltpu.VMEM((K, D), jnp.float32),
            pltpu.VMEM((RPT,), jnp.float32),
            pltpu.VMEM_SHARED((NP,), jnp.float32),
            pltpu.VMEM_SHARED((NP,), jnp.float32),
            pltpu.VMEM_SHARED((NP, D), jnp.float32),
            pltpu.VMEM_SHARED((NP,), jnp.float32),
            pltpu.SemaphoreType.DMA,
        ],
    )
    return fn(xp, a_src, a_dst, src, dst)


def _ext_weights(w, att_s, att_d):
    us = w @ att_s.reshape(D)
    ud = w @ att_d.reshape(D)
    pad = jnp.zeros((D, 2 * D - D - 2), jnp.float32)
    return jnp.concatenate([w, us[:, None], ud[:, None], pad], axis=1)


def kernel(x, edge_index, W1, att_src1, att_dst1, b1,
           W2, att_src2, att_dst2, b2):
    src = edge_index[0]
    dst = edge_index[1]
    w1e = _ext_weights(W1, att_src1, att_dst1)
    w2e = _ext_weights(W2, att_src2, att_dst2)

    def _padded(col):
        return jnp.zeros((NP,), jnp.float32).at[:N].set(col)

    xp1e = _matmul_ext(x, w1e)
    xp1 = xp1e[:, :D]
    a_s1 = _padded(xp1e[:, D])
    a_d1 = _padded(xp1e[:, D + 1])
    acc1, den1 = _edge_pass(xp1, a_s1, a_d1, src, dst)

    xp2e = _combine_matmul(acc1[0, :N], acc1[1, :N],
                           den1[0, :N, None], den1[1, :N, None],
                           b1[None, :], w2e)
    xp2 = xp2e[:, :D]
    a_s2 = _padded(xp2e[:, D])
    a_d2 = _padded(xp2e[:, D + 1])
    acc2, den2 = _edge_pass(xp2, a_s2, a_d2, src, dst)

    out = _combine(acc2[0, :N], acc2[1, :N],
                   den2[0, :N, None], den2[1, :N, None], b2[None, :])
    return out
